# Initial kernel scaffold; baseline (speedup 1.0000x reference)
import functools

import jax
import jax.numpy as jnp
from jax import lax
from jax.experimental import pallas as pl
from jax.experimental.pallas import tpu as pltpu
from jax.experimental.pallas import tpu_sc as plsc

N = 10000
E = 160000
D = 256
HALF = 128
NC, NS = 2, 16
NW = NC * NS
KE = 128
NCHUNK = E // KE
NB_PAD = 10240


def _vsm():
    return plsc.VectorSubcoreMesh(
        core_axis_name="c", subcore_axis_name="s", num_cores=NC, num_subcores=NS)


def _bank_gather(table, idx_pad):
    nchunk = NB_PAD // KE

    @functools.partial(
        pl.kernel,
        out_type=jax.ShapeDtypeStruct((NB_PAD, 16), jnp.float32),
        mesh=_vsm(),
        scratch_types=[
            pltpu.VMEM((KE,), jnp.int32),
            pltpu.VMEM((KE, 16), jnp.float32),
            pltpu.SemaphoreType.DMA,
        ],
    )
    def k(table_hbm, idx_hbm, out_hbm, idx_v, rows_v, sem):
        cid = lax.axis_index("c")
        sid = lax.axis_index("s")
        wid = sid * NC + cid

        def body(j, _):
            kk = wid + NW * j

            @pl.when(kk < nchunk)
            def _():
                base = kk * KE
                pltpu.sync_copy(idx_hbm.at[pl.ds(base, KE)], idx_v)
                pltpu.async_copy(table_hbm.at[idx_v], rows_v, sem).wait()
                pltpu.sync_copy(rows_v, out_hbm.at[pl.ds(base, KE)])
            return 0

        lax.fori_loop(0, (nchunk + NW - 1) // NW, body, 0)

    return k(table, idx_pad)


def _segsum2(xin2, xout2, src, dst):
    out_sds = jax.ShapeDtypeStruct((NC * N, HALF), jnp.float32)
    nzero = N // 16
    rows_per_tile = N // NS

    @functools.partial(
        pl.kernel,
        out_type=(out_sds, out_sds),
        mesh=_vsm(),
        scratch_types=[
            pltpu.VMEM((KE,), jnp.int32),
            pltpu.VMEM((KE,), jnp.int32),
            pltpu.VMEM((KE, HALF), jnp.float32),
            pltpu.VMEM((16, HALF), jnp.float32),
            pltpu.VMEM_SHARED((N, HALF), jnp.float32),
            pltpu.SemaphoreType.DMA,
        ],
    )
    def k(xin_hbm, xout_hbm, src_hbm, dst_hbm, aggin_hbm, aggout_hbm,
          gidx_v, sidx_v, rows_v, zbuf_v, acc_sh, sem):
        cid = lax.axis_index("c")
        sid = lax.axis_index("s")
        row_off = cid * N

        for r in range(16):
            for c8 in range(HALF // 16):
                zbuf_v[r, pl.ds(c8 * 16, 16)] = jnp.zeros((16,), jnp.float32)

        def zero_acc():
            def zb(j, _):
                kk = sid + NS * j

                @pl.when(kk < nzero)
                def _():
                    pltpu.sync_copy(zbuf_v, acc_sh.at[pl.ds(kk * 16, 16)])
                return 0

            lax.fori_loop(0, (nzero + NS - 1) // NS, zb, 0)

        def accumulate(tab_hbm, gat_hbm, sct_hbm):
            def eb(j, _):
                kk = sid + NS * j

                @pl.when(kk < NCHUNK)
                def _():
                    base = kk * KE
                    pltpu.sync_copy(gat_hbm.at[pl.ds(base, KE)], gidx_v)
                    for c16 in range(KE // 16):
                        sl = pl.ds(c16 * 16, 16)
                        gidx_v[sl] = gidx_v[sl] + row_off
                    pltpu.sync_copy(sct_hbm.at[pl.ds(base, KE)], sidx_v)
                    pltpu.async_copy(tab_hbm.at[gidx_v], rows_v, sem).wait()
                    pltpu.sync_copy(rows_v, acc_sh.at[sidx_v], add=True)
                return 0

            lax.fori_loop(0, (NCHUNK + NS - 1) // NS, eb, 0)

        def writeout(out_hbm):
            r0 = sid * rows_per_tile
            pltpu.sync_copy(acc_sh.at[pl.ds(r0, rows_per_tile)],
                            out_hbm.at[pl.ds(row_off + r0, rows_per_tile)])

        zero_acc()
        plsc.subcore_barrier()
        accumulate(xin_hbm, src_hbm, dst_hbm)
        plsc.subcore_barrier()
        writeout(aggin_hbm)
        plsc.subcore_barrier()
        zero_acc()
        plsc.subcore_barrier()
        accumulate(xout_hbm, dst_hbm, src_hbm)
        plsc.subcore_barrier()
        writeout(aggout_hbm)

    return k(xin2, xout2, src, dst)


def _edge_gather(hf, src, dst):
    out_sds = jax.ShapeDtypeStruct((E, D), jnp.float32)

    @functools.partial(
        pl.kernel,
        out_type=(out_sds, out_sds),
        mesh=_vsm(),
        scratch_types=[
            pltpu.VMEM((KE,), jnp.int32),
            pltpu.VMEM((KE, D), jnp.float32),
            pltpu.SemaphoreType.DMA,
        ],
    )
    def k(hf_hbm, src_hbm, dst_hbm, sh_hbm, dh_hbm, idx_v, rows_v, sem):
        cid = lax.axis_index("c")
        sid = lax.axis_index("s")
        wid = sid * NC + cid

        def body(j, _):
            kk = wid + NW * j

            @pl.when(kk < NCHUNK)
            def _():
                base = kk * KE
                pltpu.sync_copy(src_hbm.at[pl.ds(base, KE)], idx_v)
                pltpu.async_copy(hf_hbm.at[idx_v], rows_v, sem).wait()
                pltpu.sync_copy(rows_v, sh_hbm.at[pl.ds(base, KE)])
                pltpu.sync_copy(dst_hbm.at[pl.ds(base, KE)], idx_v)
                pltpu.async_copy(hf_hbm.at[idx_v], rows_v, sem).wait()
                pltpu.sync_copy(rows_v, dh_hbm.at[pl.ds(base, KE)])
            return 0

        lax.fori_loop(0, (NCHUNK + NW - 1) // NW, body, 0)

    return k(hf, src, dst)


BN = 400


def _node_proj1(nf, be, WsA, WsB, WiA, WiB, WoA, WoB, b1):
    def body(nf_r, be_r, wsa, wsb, wia, wib, woa, wob, b1_r,
             xs_r, xin_r, xout_r):
        a = nf_r[...]
        b = be_r[...]
        dot = lambda x, w: jnp.dot(x, w[...], preferred_element_type=jnp.float32)
        xs_r[...] = dot(a, wsa) + dot(b, wsb) + b1_r[...]
        xin_r[...] = dot(a, wia) + dot(b, wib)
        xout_r[...] = dot(a, woa) + dot(b, wob)

    full = lambda s: pl.BlockSpec(s, lambda i: (0,) * len(s))
    blk = pl.BlockSpec((BN, D), lambda i: (i, 0))
    sds = jax.ShapeDtypeStruct((N, D), jnp.float32)
    return pl.pallas_call(
        body,
        grid=(N // BN,),
        in_specs=[blk, pl.BlockSpec((BN, 16), lambda i: (i, 0)),
                  full((D, D)), full((16, D)), full((D, D)), full((16, D)),
                  full((D, D)), full((16, D)), full((1, D))],
        out_specs=[blk, blk, blk],
        out_shape=[sds, sds, sds],
    )(nf, be, WsA, WsB, WiA, WiB, WoA, WoB, b1)


def _ln_proj2(xs, aggin, aggout, g, b, W2sT, b2, W2iT, W2oT):
    def body(xs_r, ai_r, ao_r, g_r, b_r, ws, b2_r, wi, wo,
             hs_r, hi_r, ho_r):
        pre = xs_r[...] + ai_r[...] + ao_r[...]
        m = jnp.mean(pre, axis=1, keepdims=True)
        v = jnp.mean((pre - m) * (pre - m), axis=1, keepdims=True)
        h = (pre - m) * lax.rsqrt(v + 1e-5) * g_r[...] + b_r[...]
        h = jnp.maximum(h, 0.0)
        dot = lambda x, w: jnp.dot(x, w[...], preferred_element_type=jnp.float32)
        hs_r[...] = dot(h, ws) + b2_r[...]
        hi_r[...] = dot(h, wi)
        ho_r[...] = dot(h, wo)

    full = lambda s: pl.BlockSpec(s, lambda i: (0,) * len(s))
    blk = pl.BlockSpec((BN, D), lambda i: (i, 0))
    sds = jax.ShapeDtypeStruct((N, D), jnp.float32)
    return pl.pallas_call(
        body,
        grid=(N // BN,),
        in_specs=[blk, blk, blk, full((1, D)), full((1, D)),
                  full((D, D)), full((1, D)), full((D, D)), full((D, D))],
        out_specs=[blk, blk, blk],
        out_shape=[sds, sds, sds],
    )(xs, aggin, aggout, g, b, W2sT, b2, W2iT, W2oT)


def _ln_final(h2s, aggin, aggout, g, b, FT, GT, HT, ct, ft):
    def body(xs_r, ai_r, ao_r, g_r, b_r, ftab_r, gtab_r, htab_r, ct_r, ftm_r,
             hf_r, woh_r):
        pre = xs_r[...] + ai_r[...] + ao_r[...]
        m = jnp.mean(pre, axis=1, keepdims=True)
        v = jnp.mean((pre - m) * (pre - m), axis=1, keepdims=True)
        h = (pre - m) * lax.rsqrt(v + 1e-5) * g_r[...] + b_r[...]
        hf_r[...] = jnp.maximum(h, 0.0)

        @pl.when(pl.program_id(0) == 0)
        def _():
            dot = lambda x, w: jnp.dot(x, w, preferred_element_type=jnp.float32)
            woh_r[...] = jnp.zeros((128, D), jnp.float32)
            woh_r[0:32, :] = dot(ct_r[...], ftab_r[...])
            woh_r[32:64, :] = dot(ct_r[...], gtab_r[...])
            woh_r[64:72, :] = dot(ftm_r[...], htab_r[...])

    full = lambda s: pl.BlockSpec(s, lambda i: (0,) * len(s))
    blk = pl.BlockSpec((BN, D), lambda i: (i, 0))
    return pl.pallas_call(
        body,
        grid=(N // BN,),
        in_specs=[blk, blk, blk, full((1, D)), full((1, D)),
                  full((8, D)), full((8, D)), full((8, D)),
                  full((32, 8)), full((8, 8))],
        out_specs=[blk, full((128, D))],
        out_shape=[jax.ShapeDtypeStruct((N, D), jnp.float32),
                   jax.ShapeDtypeStruct((128, D), jnp.float32)],
    )(h2s, aggin, aggout, g, b, FT, GT, HT, ct, ft)


BE = 640


def _edge_mlp(SH, DH, sm, AT, BT, CT, DT, Woh, Wsm, Wm2T, bm2, wm3, bm3):
    def body(sh_r, dh_r, sm_r, at, bt, ct_, dt, woh, wsm, w2, b2, w3, b3,
             out_r):
        sh = sh_r[...]
        dh = dh_r[...]
        sm_ = sm_r[...]
        dot = lambda x, w: jnp.dot(x, w[...], preferred_element_type=jnp.float32)
        acc = dot(sh, at) + dot(dh, bt)
        acc += dot(jnp.abs(sh - dh), ct_)
        acc += dot(sh * dh, dt)
        lanes = lax.broadcasted_iota(jnp.float32, (BE, 128), 1)
        cs = sm_[:, 8:9]
        cr = sm_[:, 9:10]
        pf = sm_[:, 10:11]
        oh = ((lanes == cs) | (lanes == 32.0 + cr) |
              (lanes == 64.0 + pf)).astype(jnp.float32)
        acc += dot(oh, woh)
        acc += dot(sm_, wsm)
        z1 = jnp.maximum(acc, 0.0)
        z2 = jnp.maximum(dot(z1, w2) + b2[...], 0.0)
        o = jnp.sum(z2 * w3[...], axis=1, keepdims=True) + b3[...]
        out_r[...] = jnp.broadcast_to(o, (BE, 8))

    full = lambda s: pl.BlockSpec(s, lambda i: (0,) * len(s))
    blk = pl.BlockSpec((BE, D), lambda i: (i, 0))
    return pl.pallas_call(
        body,
        grid=(E // BE,),
        in_specs=[blk, blk, pl.BlockSpec((BE, 16), lambda i: (i, 0)),
                  full((D, D)), full((D, D)), full((D, D)), full((D, D)),
                  full((128, D)), full((16, D)),
                  full((D, 128)), full((1, 128)), full((1, 128)), full((1, 1))],
        out_specs=[pl.BlockSpec((BE, 8), lambda i: (i, 0))],
        out_shape=[jax.ShapeDtypeStruct((E, 8), jnp.float32)],
    )(SH, DH, sm, AT, BT, CT, DT, Woh, Wsm, Wm2T, bm2, wm3, bm3)


def _split_half(x):
    return x.reshape(N, 2, HALF).transpose(1, 0, 2).reshape(2 * N, HALF)


def _unsplit_half(x2):
    return x2.reshape(2, N, HALF).transpose(1, 0, 2).reshape(N, D)


def kernel(node_features, edge_numeric, node_bank_ids, edge_src, edge_dst,
           edge_sent_currency, edge_recv_currency, edge_payment_format,
           bank_table, currency_table, format_table,
           W1_self, b1_self, W1_in, W1_out, g1, be1,
           W2_self, b2_self, W2_in, W2_out, g2, be2,
           Wm1, bm1, Wm2, bm2, Wm3, bm3):
    f32 = jnp.float32
    src = edge_src.astype(jnp.int32)
    dst = edge_dst.astype(jnp.int32)

    bids_pad = jnp.pad(node_bank_ids.astype(jnp.int32), (0, NB_PAD - N))
    be_emb = _bank_gather(bank_table, bids_pad)[:N]

    row = lambda v: v.reshape(1, -1)
    xs, xin, xout = _node_proj1(
        node_features, be_emb,
        W1_self[:, :D].T, W1_self[:, D:].T,
        W1_in[:, :D].T, W1_in[:, D:].T,
        W1_out[:, :D].T, W1_out[:, D:].T,
        row(b1_self))

    aggin2, aggout2 = _segsum2(_split_half(xin), _split_half(xout), src, dst)

    h2s, h2i, h2o = _ln_proj2(
        xs, _unsplit_half(aggin2), _unsplit_half(aggout2),
        row(g1), row(be1), W2_self.T, row(b2_self), W2_in.T, W2_out.T)

    agg2in2, agg2out2 = _segsum2(_split_half(h2i), _split_half(h2o), src, dst)

    A = Wm1[:, 0:256]
    B = Wm1[:, 256:512]
    C = Wm1[:, 512:768]
    Dm = Wm1[:, 768:1024]
    Et = Wm1[:, 1024:1030].T
    FT = Wm1[:, 1030:1038].T
    GT = Wm1[:, 1038:1046].T
    HT = Wm1[:, 1046:1054].T
    hf, Woh = _ln_final(
        h2s, _unsplit_half(agg2in2), _unsplit_half(agg2out2),
        row(g2), row(be2), FT, GT, HT, currency_table, format_table)

    SH, DH = _edge_gather(hf, src, dst)

    sm = jnp.concatenate([
        edge_numeric.astype(f32),
        jnp.ones((E, 1), f32),
        jnp.zeros((E, 1), f32),
        edge_sent_currency.astype(f32).reshape(E, 1),
        edge_recv_currency.astype(f32).reshape(E, 1),
        edge_payment_format.astype(f32).reshape(E, 1),
        jnp.zeros((E, 5), f32),
    ], axis=1)
    Wsm = jnp.concatenate([Et, bm1.reshape(1, D), jnp.zeros((9, D), f32)], axis=0)
    wm3 = Wm3.reshape(1, -1)
    out8 = _edge_mlp(SH, DH, sm,
                     A.T, B.T, C.T, Dm.T, Woh, Wsm,
                     Wm2.T, bm2.reshape(1, -1), wm3, bm3.reshape(1, 1))
    return out8[:, 0]


# speedup vs baseline: 2.6053x; 2.6053x over previous
import functools

import jax
import jax.numpy as jnp
from jax import lax
from jax.experimental import pallas as pl
from jax.experimental.pallas import tpu as pltpu
from jax.experimental.pallas import tpu_sc as plsc

N = 10000
E = 160000
D = 256
HALF = 128
NC, NS = 2, 16
NW = NC * NS
KE = 128
NCHUNK = E // KE
NB_PAD = 10240


def _vsm():
    return plsc.VectorSubcoreMesh(
        core_axis_name="c", subcore_axis_name="s", num_cores=NC, num_subcores=NS)


def _bank_gather(table, idx_pad):
    nchunk = NB_PAD // KE

    @functools.partial(
        pl.kernel,
        out_type=jax.ShapeDtypeStruct((NB_PAD, HALF), jnp.float32),
        mesh=_vsm(),
        scratch_types=[
            pltpu.VMEM((KE,), jnp.int32),
            pltpu.VMEM((KE, HALF), jnp.float32),
            pltpu.SemaphoreType.DMA,
        ],
    )
    def k(table_hbm, idx_hbm, out_hbm, idx_v, rows_v, sem):
        cid = lax.axis_index("c")
        sid = lax.axis_index("s")
        wid = sid * NC + cid

        def body(j, _):
            kk = wid + NW * j

            @pl.when(kk < nchunk)
            def _():
                base = kk * KE
                pltpu.sync_copy(idx_hbm.at[pl.ds(base, KE)], idx_v)
                pltpu.async_copy(table_hbm.at[idx_v], rows_v, sem).wait()
                pltpu.sync_copy(rows_v, out_hbm.at[pl.ds(base, KE)])
            return 0

        lax.fori_loop(0, (nchunk + NW - 1) // NW, body, 0)

    return k(table, idx_pad)


def _segsum2(xin2, xout2, src, dst):
    out_sds = jax.ShapeDtypeStruct((NC * N, HALF), jnp.float32)
    nzero = N // 16
    rows_per_tile = N // NS

    @functools.partial(
        pl.kernel,
        out_type=(out_sds, out_sds),
        mesh=_vsm(),
        scratch_types=[
            pltpu.VMEM((KE,), jnp.int32),
            pltpu.VMEM((KE,), jnp.int32),
            pltpu.VMEM((KE, HALF), jnp.float32),
            pltpu.VMEM((16, HALF), jnp.float32),
            pltpu.VMEM_SHARED((N, HALF), jnp.float32),
            pltpu.SemaphoreType.DMA,
        ],
    )
    def k(xin_hbm, xout_hbm, src_hbm, dst_hbm, aggin_hbm, aggout_hbm,
          gidx_v, sidx_v, rows_v, zbuf_v, acc_sh, sem):
        cid = lax.axis_index("c")
        sid = lax.axis_index("s")
        row_off = cid * N

        for r in range(16):
            for c8 in range(HALF // 16):
                zbuf_v[r, pl.ds(c8 * 16, 16)] = jnp.zeros((16,), jnp.float32)

        def zero_acc():
            def zb(j, _):
                kk = sid + NS * j

                @pl.when(kk < nzero)
                def _():
                    pltpu.sync_copy(zbuf_v, acc_sh.at[pl.ds(kk * 16, 16)])
                return 0

            lax.fori_loop(0, (nzero + NS - 1) // NS, zb, 0)

        def accumulate(tab_hbm, gat_hbm, sct_hbm):
            def eb(j, _):
                kk = sid + NS * j

                @pl.when(kk < NCHUNK)
                def _():
                    base = kk * KE
                    pltpu.sync_copy(gat_hbm.at[pl.ds(base, KE)], gidx_v)
                    for c16 in range(KE // 16):
                        sl = pl.ds(c16 * 16, 16)
                        gidx_v[sl] = gidx_v[sl] + row_off
                    pltpu.sync_copy(sct_hbm.at[pl.ds(base, KE)], sidx_v)
                    pltpu.async_copy(tab_hbm.at[gidx_v], rows_v, sem).wait()
                    pltpu.sync_copy(rows_v, acc_sh.at[sidx_v], add=True)
                return 0

            lax.fori_loop(0, (NCHUNK + NS - 1) // NS, eb, 0)

        def writeout(out_hbm):
            def wb(j, _):
                kk = sid + NS * j

                @pl.when(kk < nzero)
                def _():
                    pltpu.sync_copy(acc_sh.at[pl.ds(kk * 16, 16)],
                                    out_hbm.at[pl.ds(row_off + kk * 16, 16)])
                return 0

            lax.fori_loop(0, (nzero + NS - 1) // NS, wb, 0)

        zero_acc()
        plsc.subcore_barrier()
        accumulate(xin_hbm, src_hbm, dst_hbm)
        plsc.subcore_barrier()
        writeout(aggin_hbm)
        plsc.subcore_barrier()
        zero_acc()
        plsc.subcore_barrier()
        accumulate(xout_hbm, dst_hbm, src_hbm)
        plsc.subcore_barrier()
        writeout(aggout_hbm)

    return k(xin2, xout2, src, dst)


def _edge_gather(hf, src, dst):
    out_sds = jax.ShapeDtypeStruct((E, D), jnp.float32)

    @functools.partial(
        pl.kernel,
        out_type=(out_sds, out_sds),
        mesh=_vsm(),
        scratch_types=[
            pltpu.VMEM((KE,), jnp.int32),
            pltpu.VMEM((KE, D), jnp.float32),
            pltpu.SemaphoreType.DMA,
        ],
    )
    def k(hf_hbm, src_hbm, dst_hbm, sh_hbm, dh_hbm, idx_v, rows_v, sem):
        cid = lax.axis_index("c")
        sid = lax.axis_index("s")
        wid = sid * NC + cid

        def body(j, _):
            kk = wid + NW * j

            @pl.when(kk < NCHUNK)
            def _():
                base = kk * KE
                pltpu.sync_copy(src_hbm.at[pl.ds(base, KE)], idx_v)
                pltpu.async_copy(hf_hbm.at[idx_v], rows_v, sem).wait()
                pltpu.sync_copy(rows_v, sh_hbm.at[pl.ds(base, KE)])
                pltpu.sync_copy(dst_hbm.at[pl.ds(base, KE)], idx_v)
                pltpu.async_copy(hf_hbm.at[idx_v], rows_v, sem).wait()
                pltpu.sync_copy(rows_v, dh_hbm.at[pl.ds(base, KE)])
            return 0

        lax.fori_loop(0, (NCHUNK + NW - 1) // NW, body, 0)

    return k(hf, src, dst)


BN = 400


def _node_proj1(nf, be, WsA, WsB, WiA, WiB, WoA, WoB, b1):
    def body(nf_r, be_r, wsa, wsb, wia, wib, woa, wob, b1_r,
             xs_r, xin_r, xout_r):
        a = nf_r[...]
        b = be_r[...]
        dot = lambda x, w: jnp.dot(x, w[...], preferred_element_type=jnp.float32)
        xs_r[...] = dot(a, wsa) + dot(b, wsb) + b1_r[...]
        xin_r[...] = dot(a, wia) + dot(b, wib)
        xout_r[...] = dot(a, woa) + dot(b, wob)

    full = lambda s: pl.BlockSpec(s, lambda i: (0,) * len(s))
    blk = pl.BlockSpec((BN, D), lambda i: (i, 0))
    sds = jax.ShapeDtypeStruct((N, D), jnp.float32)
    return pl.pallas_call(
        body,
        grid=(N // BN,),
        in_specs=[blk, pl.BlockSpec((BN, 16), lambda i: (i, 0)),
                  full((D, D)), full((16, D)), full((D, D)), full((16, D)),
                  full((D, D)), full((16, D)), full((1, D))],
        out_specs=[blk, blk, blk],
        out_shape=[sds, sds, sds],
    )(nf, be, WsA, WsB, WiA, WiB, WoA, WoB, b1)


def _ln_proj2(xs, aggin, aggout, g, b, W2sT, b2, W2iT, W2oT):
    def body(xs_r, ai_r, ao_r, g_r, b_r, ws, b2_r, wi, wo,
             hs_r, hi_r, ho_r):
        pre = xs_r[...] + ai_r[...] + ao_r[...]
        m = jnp.mean(pre, axis=1, keepdims=True)
        v = jnp.mean((pre - m) * (pre - m), axis=1, keepdims=True)
        h = (pre - m) * lax.rsqrt(v + 1e-5) * g_r[...] + b_r[...]
        h = jnp.maximum(h, 0.0)
        dot = lambda x, w: jnp.dot(x, w[...], preferred_element_type=jnp.float32)
        hs_r[...] = dot(h, ws) + b2_r[...]
        hi_r[...] = dot(h, wi)
        ho_r[...] = dot(h, wo)

    full = lambda s: pl.BlockSpec(s, lambda i: (0,) * len(s))
    blk = pl.BlockSpec((BN, D), lambda i: (i, 0))
    sds = jax.ShapeDtypeStruct((N, D), jnp.float32)
    return pl.pallas_call(
        body,
        grid=(N // BN,),
        in_specs=[blk, blk, blk, full((1, D)), full((1, D)),
                  full((D, D)), full((1, D)), full((D, D)), full((D, D))],
        out_specs=[blk, blk, blk],
        out_shape=[sds, sds, sds],
    )(xs, aggin, aggout, g, b, W2sT, b2, W2iT, W2oT)


def _ln_final(h2s, aggin, aggout, g, b, FT, GT, HT, ct, ft):
    def body(xs_r, ai_r, ao_r, g_r, b_r, ftab_r, gtab_r, htab_r, ct_r, ftm_r,
             hf_r, woh_r):
        pre = xs_r[...] + ai_r[...] + ao_r[...]
        m = jnp.mean(pre, axis=1, keepdims=True)
        v = jnp.mean((pre - m) * (pre - m), axis=1, keepdims=True)
        h = (pre - m) * lax.rsqrt(v + 1e-5) * g_r[...] + b_r[...]
        hf_r[...] = jnp.maximum(h, 0.0)

        @pl.when(pl.program_id(0) == 0)
        def _():
            dot = lambda x, w: jnp.dot(x, w, preferred_element_type=jnp.float32)
            woh_r[...] = jnp.zeros((128, D), jnp.float32)
            woh_r[0:32, :] = dot(ct_r[...], ftab_r[...])
            woh_r[32:64, :] = dot(ct_r[...], gtab_r[...])
            woh_r[64:72, :] = dot(ftm_r[...], htab_r[...])

    full = lambda s: pl.BlockSpec(s, lambda i: (0,) * len(s))
    blk = pl.BlockSpec((BN, D), lambda i: (i, 0))
    return pl.pallas_call(
        body,
        grid=(N // BN,),
        in_specs=[blk, blk, blk, full((1, D)), full((1, D)),
                  full((8, D)), full((8, D)), full((8, D)),
                  full((32, 8)), full((8, 8))],
        out_specs=[blk, full((128, D))],
        out_shape=[jax.ShapeDtypeStruct((N, D), jnp.float32),
                   jax.ShapeDtypeStruct((128, D), jnp.float32)],
    )(h2s, aggin, aggout, g, b, FT, GT, HT, ct, ft)


BE = 640


def _edge_mlp(SH, DH, sm, AT, BT, CT, DT, Woh, Wsm, Wm2T, bm2, wm3, bm3):
    def body(sh_r, dh_r, sm_r, at, bt, ct_, dt, woh, wsm, w2, b2, w3, b3,
             out_r):
        sh = sh_r[...]
        dh = dh_r[...]
        sm_ = sm_r[...]
        dot = lambda x, w: jnp.dot(x, w[...], preferred_element_type=jnp.float32)
        acc = dot(sh, at) + dot(dh, bt)
        acc += dot(jnp.abs(sh - dh), ct_)
        acc += dot(sh * dh, dt)
        lanes = lax.broadcasted_iota(jnp.int32, (BE, 128), 1)
        cs = sm_[:, 8:9].astype(jnp.int32)
        cr = sm_[:, 9:10].astype(jnp.int32)
        pf = sm_[:, 10:11].astype(jnp.int32)
        oh = ((lanes == cs) | (lanes == 32 + cr) |
              (lanes == 64 + pf)).astype(jnp.float32)
        acc += dot(oh, woh)
        acc += dot(sm_, wsm)
        z1 = jnp.maximum(acc, 0.0)
        z2 = jnp.maximum(dot(z1, w2) + b2[...], 0.0)
        o = jnp.sum(z2 * w3[...], axis=1, keepdims=True) + b3[...]
        out_r[...] = jnp.broadcast_to(o, (BE, 8))

    full = lambda s: pl.BlockSpec(s, lambda i: (0,) * len(s))
    blk = pl.BlockSpec((BE, D), lambda i: (i, 0))
    return pl.pallas_call(
        body,
        grid=(E // BE,),
        in_specs=[blk, blk, pl.BlockSpec((BE, 16), lambda i: (i, 0)),
                  full((D, D)), full((D, D)), full((D, D)), full((D, D)),
                  full((128, D)), full((16, D)),
                  full((D, 128)), full((1, 128)), full((1, 128)), full((1, 1))],
        out_specs=[pl.BlockSpec((BE, 8), lambda i: (i, 0))],
        out_shape=[jax.ShapeDtypeStruct((E, 8), jnp.float32)],
    )(SH, DH, sm, AT, BT, CT, DT, Woh, Wsm, Wm2T, bm2, wm3, bm3)[0]


def _split_half(x):
    return x.reshape(N, 2, HALF).transpose(1, 0, 2).reshape(2 * N, HALF)


def _unsplit_half(x2):
    return x2.reshape(2, N, HALF).transpose(1, 0, 2).reshape(N, D)


def kernel(node_features, edge_numeric, node_bank_ids, edge_src, edge_dst,
           edge_sent_currency, edge_recv_currency, edge_payment_format,
           bank_table, currency_table, format_table,
           W1_self, b1_self, W1_in, W1_out, g1, be1,
           W2_self, b2_self, W2_in, W2_out, g2, be2,
           Wm1, bm1, Wm2, bm2, Wm3, bm3):
    f32 = jnp.float32
    src = edge_src.astype(jnp.int32)
    dst = edge_dst.astype(jnp.int32)

    bids_pad = jnp.pad(node_bank_ids.astype(jnp.int32), (0, NB_PAD - N))
    bank_pad = jnp.pad(bank_table, ((0, 0), (0, HALF - 16)))
    be_emb = _bank_gather(bank_pad, bids_pad)[:N, :16]

    row = lambda v: v.reshape(1, -1)
    xs, xin, xout = _node_proj1(
        node_features, be_emb,
        W1_self[:, :D].T, W1_self[:, D:].T,
        W1_in[:, :D].T, W1_in[:, D:].T,
        W1_out[:, :D].T, W1_out[:, D:].T,
        row(b1_self))

    aggin2, aggout2 = _segsum2(_split_half(xin), _split_half(xout), src, dst)

    h2s, h2i, h2o = _ln_proj2(
        xs, _unsplit_half(aggin2), _unsplit_half(aggout2),
        row(g1), row(be1), W2_self.T, row(b2_self), W2_in.T, W2_out.T)

    agg2in2, agg2out2 = _segsum2(_split_half(h2i), _split_half(h2o), src, dst)

    A = Wm1[:, 0:256]
    B = Wm1[:, 256:512]
    C = Wm1[:, 512:768]
    Dm = Wm1[:, 768:1024]
    Et = Wm1[:, 1024:1030].T
    FT = Wm1[:, 1030:1038].T
    GT = Wm1[:, 1038:1046].T
    HT = Wm1[:, 1046:1054].T
    hf, Woh = _ln_final(
        h2s, _unsplit_half(agg2in2), _unsplit_half(agg2out2),
        row(g2), row(be2), FT, GT, HT, currency_table, format_table)

    SH, DH = _edge_gather(hf, src, dst)

    sm = jnp.concatenate([
        edge_numeric.astype(f32),
        jnp.ones((E, 1), f32),
        jnp.zeros((E, 1), f32),
        edge_sent_currency.astype(f32).reshape(E, 1),
        edge_recv_currency.astype(f32).reshape(E, 1),
        edge_payment_format.astype(f32).reshape(E, 1),
        jnp.zeros((E, 5), f32),
    ], axis=1)
    Wsm = jnp.concatenate([Et, bm1.reshape(1, D), jnp.zeros((9, D), f32)], axis=0)
    wm3 = Wm3.reshape(1, -1)
    out8 = _edge_mlp(SH, DH, sm,
                     A.T, B.T, C.T, Dm.T, Woh, Wsm,
                     Wm2.T, bm2.reshape(1, -1), wm3, bm3.reshape(1, 1))
    return out8[:, 0]


# speedup vs baseline: 3.3274x; 1.2772x over previous
import functools

import jax
import jax.numpy as jnp
from jax import lax
from jax.experimental import pallas as pl
from jax.experimental.pallas import tpu as pltpu
from jax.experimental.pallas import tpu_sc as plsc

N = 10000
E = 160000
D = 256
HALF = 128
NC, NS = 2, 16
NW = NC * NS
KE = 128
NCHUNK = E // KE
NB_PAD = 10240


def _vsm():
    return plsc.VectorSubcoreMesh(
        core_axis_name="c", subcore_axis_name="s", num_cores=NC, num_subcores=NS)


def _bank_gather(table, idx_pad):
    nchunk = NB_PAD // KE

    @functools.partial(
        pl.kernel,
        out_type=jax.ShapeDtypeStruct((NB_PAD, HALF), jnp.float32),
        mesh=_vsm(),
        scratch_types=[
            pltpu.VMEM((KE,), jnp.int32),
            pltpu.VMEM((KE, HALF), jnp.float32),
            pltpu.SemaphoreType.DMA,
        ],
    )
    def k(table_hbm, idx_hbm, out_hbm, idx_v, rows_v, sem):
        cid = lax.axis_index("c")
        sid = lax.axis_index("s")
        wid = sid * NC + cid

        def body(j, _):
            kk = wid + NW * j

            @pl.when(kk < nchunk)
            def _():
                base = kk * KE
                pltpu.sync_copy(idx_hbm.at[pl.ds(base, KE)], idx_v)
                pltpu.async_copy(table_hbm.at[idx_v], rows_v, sem).wait()
                pltpu.sync_copy(rows_v, out_hbm.at[pl.ds(base, KE)])
            return 0

        lax.fori_loop(0, (nchunk + NW - 1) // NW, body, 0)

    return k(table, idx_pad)


def _segsum2(xin2, xout2, src, dst):
    out_sds = jax.ShapeDtypeStruct((NC * N, HALF), jnp.float32)
    nzero = N // 16
    rows_per_tile = N // NS

    @functools.partial(
        pl.kernel,
        out_type=(out_sds, out_sds),
        mesh=_vsm(),
        scratch_types=[
            pltpu.VMEM((KE,), jnp.int32),
            pltpu.VMEM((KE,), jnp.int32),
            pltpu.VMEM((KE,), jnp.int32),
            pltpu.VMEM((KE,), jnp.int32),
            pltpu.VMEM((KE, HALF), jnp.float32),
            pltpu.VMEM((KE, HALF), jnp.float32),
            pltpu.VMEM((16, HALF), jnp.float32),
            pltpu.VMEM_SHARED((N, HALF), jnp.float32),
            pltpu.SemaphoreType.DMA,
            pltpu.SemaphoreType.DMA,
        ],
    )
    def k(xin_hbm, xout_hbm, src_hbm, dst_hbm, aggin_hbm, aggout_hbm,
          gidx0_v, gidx1_v, sidx0_v, sidx1_v, rows0_v, rows1_v,
          zbuf_v, acc_sh, sem0, sem1):
        cid = lax.axis_index("c")
        sid = lax.axis_index("s")
        row_off = cid * N

        for r in range(16):
            for c8 in range(HALF // 16):
                zbuf_v[r, pl.ds(c8 * 16, 16)] = jnp.zeros((16,), jnp.float32)

        def zero_acc():
            def zb(j, _):
                kk = sid + NS * j

                @pl.when(kk < nzero)
                def _():
                    pltpu.sync_copy(zbuf_v, acc_sh.at[pl.ds(kk * 16, 16)])
                return 0

            lax.fori_loop(0, (nzero + NS - 1) // NS, zb, 0)

        def accumulate(tab_hbm, gat_hbm, sct_hbm):
            def start(j, gidx_v, sidx_v, rows_v, sem):
                kk = sid + NS * j

                @pl.when(kk < NCHUNK)
                def _():
                    base = kk * KE
                    pltpu.sync_copy(gat_hbm.at[pl.ds(base, KE)], gidx_v)
                    for c16 in range(KE // 16):
                        sl = pl.ds(c16 * 16, 16)
                        gidx_v[sl] = gidx_v[sl] + row_off
                    pltpu.sync_copy(sct_hbm.at[pl.ds(base, KE)], sidx_v)
                    pltpu.async_copy(tab_hbm.at[gidx_v], rows_v, sem)

            def finish(j, gidx_v, sidx_v, rows_v, sem):
                kk = sid + NS * j

                @pl.when(kk < NCHUNK)
                def _():
                    pltpu.make_async_copy(tab_hbm.at[gidx_v], rows_v, sem).wait()
                    pltpu.sync_copy(rows_v, acc_sh.at[sidx_v], add=True)

            start(0, gidx0_v, sidx0_v, rows0_v, sem0)

            def outer(jj, _):
                j0 = 2 * jj
                start(j0 + 1, gidx1_v, sidx1_v, rows1_v, sem1)
                finish(j0, gidx0_v, sidx0_v, rows0_v, sem0)
                start(j0 + 2, gidx0_v, sidx0_v, rows0_v, sem0)
                finish(j0 + 1, gidx1_v, sidx1_v, rows1_v, sem1)
                return 0

            lax.fori_loop(0, 40, outer, 0)

        def writeout(out_hbm):
            def wb(j, _):
                kk = sid + NS * j

                @pl.when(kk < nzero)
                def _():
                    pltpu.sync_copy(acc_sh.at[pl.ds(kk * 16, 16)],
                                    out_hbm.at[pl.ds(row_off + kk * 16, 16)])
                return 0

            lax.fori_loop(0, (nzero + NS - 1) // NS, wb, 0)

        zero_acc()
        plsc.subcore_barrier()
        accumulate(xin_hbm, src_hbm, dst_hbm)
        plsc.subcore_barrier()
        writeout(aggin_hbm)
        plsc.subcore_barrier()
        zero_acc()
        plsc.subcore_barrier()
        accumulate(xout_hbm, dst_hbm, src_hbm)
        plsc.subcore_barrier()
        writeout(aggout_hbm)

    return k(xin2, xout2, src, dst)


def _edge_gather(hf, src, dst):
    out_sds = jax.ShapeDtypeStruct((E, D), jnp.float32)

    @functools.partial(
        pl.kernel,
        out_type=(out_sds, out_sds),
        mesh=_vsm(),
        scratch_types=[
            pltpu.VMEM((KE,), jnp.int32),
            pltpu.VMEM((KE,), jnp.int32),
            pltpu.VMEM((KE, D), jnp.float32),
            pltpu.VMEM((KE, D), jnp.float32),
            pltpu.SemaphoreType.DMA,
            pltpu.SemaphoreType.DMA,
        ],
    )
    def k(hf_hbm, src_hbm, dst_hbm, sh_hbm, dh_hbm,
          idx0_v, idx1_v, rows0_v, rows1_v, sem0, sem1):
        cid = lax.axis_index("c")
        sid = lax.axis_index("s")
        wid = sid * NC + cid

        def start(eidx_hbm, j, idx_v, rows_v, sem):
            kk = wid + NW * j

            @pl.when(kk < NCHUNK)
            def _():
                pltpu.sync_copy(eidx_hbm.at[pl.ds(kk * KE, KE)], idx_v)
                pltpu.async_copy(hf_hbm.at[idx_v], rows_v, sem)

        def finish(out_hbm, j, idx_v, rows_v, sem):
            kk = wid + NW * j

            @pl.when(kk < NCHUNK)
            def _():
                pltpu.make_async_copy(hf_hbm.at[idx_v], rows_v, sem).wait()
                pltpu.sync_copy(rows_v, out_hbm.at[pl.ds(kk * KE, KE)])

        start(src_hbm, 0, idx0_v, rows0_v, sem0)

        def body(j, _):
            start(dst_hbm, j, idx1_v, rows1_v, sem1)
            finish(sh_hbm, j, idx0_v, rows0_v, sem0)
            start(src_hbm, j + 1, idx0_v, rows0_v, sem0)
            finish(dh_hbm, j, idx1_v, rows1_v, sem1)
            return 0

        lax.fori_loop(0, (NCHUNK + NW - 1) // NW, body, 0)

    return k(hf, src, dst)


BN = 400


def _node_proj1(nf, be, WsA, WsB, WiA, WiB, WoA, WoB, b1):
    def body(nf_r, be_r, wsa, wsb, wia, wib, woa, wob, b1_r,
             xs_r, xin_r, xout_r):
        a = nf_r[...]
        b = be_r[...]
        dot = lambda x, w: jnp.dot(x, w[...], preferred_element_type=jnp.float32)
        xs_r[...] = dot(a, wsa) + dot(b, wsb) + b1_r[...]
        xin_r[...] = dot(a, wia) + dot(b, wib)
        xout_r[...] = dot(a, woa) + dot(b, wob)

    full = lambda s: pl.BlockSpec(s, lambda i: (0,) * len(s))
    blk = pl.BlockSpec((BN, D), lambda i: (i, 0))
    sds = jax.ShapeDtypeStruct((N, D), jnp.float32)
    return pl.pallas_call(
        body,
        grid=(N // BN,),
        in_specs=[blk, pl.BlockSpec((BN, 16), lambda i: (i, 0)),
                  full((D, D)), full((16, D)), full((D, D)), full((16, D)),
                  full((D, D)), full((16, D)), full((1, D))],
        out_specs=[blk, blk, blk],
        out_shape=[sds, sds, sds],
    )(nf, be, WsA, WsB, WiA, WiB, WoA, WoB, b1)


def _ln_proj2(xs, aggin, aggout, g, b, W2sT, b2, W2iT, W2oT):
    def body(xs_r, ai_r, ao_r, g_r, b_r, ws, b2_r, wi, wo,
             hs_r, hi_r, ho_r):
        pre = xs_r[...] + ai_r[...] + ao_r[...]
        m = jnp.mean(pre, axis=1, keepdims=True)
        v = jnp.mean((pre - m) * (pre - m), axis=1, keepdims=True)
        h = (pre - m) * lax.rsqrt(v + 1e-5) * g_r[...] + b_r[...]
        h = jnp.maximum(h, 0.0)
        dot = lambda x, w: jnp.dot(x, w[...], preferred_element_type=jnp.float32)
        hs_r[...] = dot(h, ws) + b2_r[...]
        hi_r[...] = dot(h, wi)
        ho_r[...] = dot(h, wo)

    full = lambda s: pl.BlockSpec(s, lambda i: (0,) * len(s))
    blk = pl.BlockSpec((BN, D), lambda i: (i, 0))
    sds = jax.ShapeDtypeStruct((N, D), jnp.float32)
    return pl.pallas_call(
        body,
        grid=(N // BN,),
        in_specs=[blk, blk, blk, full((1, D)), full((1, D)),
                  full((D, D)), full((1, D)), full((D, D)), full((D, D))],
        out_specs=[blk, blk, blk],
        out_shape=[sds, sds, sds],
    )(xs, aggin, aggout, g, b, W2sT, b2, W2iT, W2oT)


def _ln_final(h2s, aggin, aggout, g, b, FT, GT, HT, ct, ft):
    def body(xs_r, ai_r, ao_r, g_r, b_r, ftab_r, gtab_r, htab_r, ct_r, ftm_r,
             hf_r, woh_r):
        pre = xs_r[...] + ai_r[...] + ao_r[...]
        m = jnp.mean(pre, axis=1, keepdims=True)
        v = jnp.mean((pre - m) * (pre - m), axis=1, keepdims=True)
        h = (pre - m) * lax.rsqrt(v + 1e-5) * g_r[...] + b_r[...]
        hf_r[...] = jnp.maximum(h, 0.0)

        @pl.when(pl.program_id(0) == 0)
        def _():
            dot = lambda x, w: jnp.dot(x, w, preferred_element_type=jnp.float32)
            woh_r[...] = jnp.zeros((128, D), jnp.float32)
            woh_r[0:32, :] = dot(ct_r[...], ftab_r[...])
            woh_r[32:64, :] = dot(ct_r[...], gtab_r[...])
            woh_r[64:72, :] = dot(ftm_r[...], htab_r[...])

    full = lambda s: pl.BlockSpec(s, lambda i: (0,) * len(s))
    blk = pl.BlockSpec((BN, D), lambda i: (i, 0))
    return pl.pallas_call(
        body,
        grid=(N // BN,),
        in_specs=[blk, blk, blk, full((1, D)), full((1, D)),
                  full((8, D)), full((8, D)), full((8, D)),
                  full((32, 8)), full((8, 8))],
        out_specs=[blk, full((128, D))],
        out_shape=[jax.ShapeDtypeStruct((N, D), jnp.float32),
                   jax.ShapeDtypeStruct((128, D), jnp.float32)],
    )(h2s, aggin, aggout, g, b, FT, GT, HT, ct, ft)


BE = 640


def _edge_mlp(SH, DH, sm, AT, BT, CT, DT, Woh, Wsm, Wm2T, bm2, wm3, bm3):
    def body(sh_r, dh_r, sm_r, at, bt, ct_, dt, woh, wsm, w2, b2, w3, b3,
             out_r):
        sh = sh_r[...]
        dh = dh_r[...]
        sm_ = sm_r[...]
        dot = lambda x, w: jnp.dot(x, w[...], preferred_element_type=jnp.float32)
        acc = dot(sh, at) + dot(dh, bt)
        acc += dot(jnp.abs(sh - dh), ct_)
        acc += dot(sh * dh, dt)
        lanes = lax.broadcasted_iota(jnp.int32, (BE, 128), 1)
        cs = sm_[:, 8:9].astype(jnp.int32)
        cr = sm_[:, 9:10].astype(jnp.int32)
        pf = sm_[:, 10:11].astype(jnp.int32)
        oh = ((lanes == cs) | (lanes == 32 + cr) |
              (lanes == 64 + pf)).astype(jnp.float32)
        acc += dot(oh, woh)
        acc += dot(sm_, wsm)
        z1 = jnp.maximum(acc, 0.0)
        z2 = jnp.maximum(dot(z1, w2) + b2[...], 0.0)
        o = jnp.sum(z2 * w3[...], axis=1, keepdims=True) + b3[...]
        out_r[...] = jnp.broadcast_to(o, (BE, 8))

    full = lambda s: pl.BlockSpec(s, lambda i: (0,) * len(s))
    blk = pl.BlockSpec((BE, D), lambda i: (i, 0))
    return pl.pallas_call(
        body,
        grid=(E // BE,),
        in_specs=[blk, blk, pl.BlockSpec((BE, 16), lambda i: (i, 0)),
                  full((D, D)), full((D, D)), full((D, D)), full((D, D)),
                  full((128, D)), full((16, D)),
                  full((D, 128)), full((1, 128)), full((1, 128)), full((1, 1))],
        out_specs=[pl.BlockSpec((BE, 8), lambda i: (i, 0))],
        out_shape=[jax.ShapeDtypeStruct((E, 8), jnp.float32)],
    )(SH, DH, sm, AT, BT, CT, DT, Woh, Wsm, Wm2T, bm2, wm3, bm3)[0]


def _split_half(x):
    return x.reshape(N, 2, HALF).transpose(1, 0, 2).reshape(2 * N, HALF)


def _unsplit_half(x2):
    return x2.reshape(2, N, HALF).transpose(1, 0, 2).reshape(N, D)


def kernel(node_features, edge_numeric, node_bank_ids, edge_src, edge_dst,
           edge_sent_currency, edge_recv_currency, edge_payment_format,
           bank_table, currency_table, format_table,
           W1_self, b1_self, W1_in, W1_out, g1, be1,
           W2_self, b2_self, W2_in, W2_out, g2, be2,
           Wm1, bm1, Wm2, bm2, Wm3, bm3):
    f32 = jnp.float32
    src = edge_src.astype(jnp.int32)
    dst = edge_dst.astype(jnp.int32)

    bids_pad = jnp.pad(node_bank_ids.astype(jnp.int32), (0, NB_PAD - N))
    bank_pad = jnp.pad(bank_table, ((0, 0), (0, HALF - 16)))
    be_emb = _bank_gather(bank_pad, bids_pad)[:N, :16]

    row = lambda v: v.reshape(1, -1)
    xs, xin, xout = _node_proj1(
        node_features, be_emb,
        W1_self[:, :D].T, W1_self[:, D:].T,
        W1_in[:, :D].T, W1_in[:, D:].T,
        W1_out[:, :D].T, W1_out[:, D:].T,
        row(b1_self))

    aggin2, aggout2 = _segsum2(_split_half(xin), _split_half(xout), src, dst)

    h2s, h2i, h2o = _ln_proj2(
        xs, _unsplit_half(aggin2), _unsplit_half(aggout2),
        row(g1), row(be1), W2_self.T, row(b2_self), W2_in.T, W2_out.T)

    agg2in2, agg2out2 = _segsum2(_split_half(h2i), _split_half(h2o), src, dst)

    A = Wm1[:, 0:256]
    B = Wm1[:, 256:512]
    C = Wm1[:, 512:768]
    Dm = Wm1[:, 768:1024]
    Et = Wm1[:, 1024:1030].T
    FT = Wm1[:, 1030:1038].T
    GT = Wm1[:, 1038:1046].T
    HT = Wm1[:, 1046:1054].T
    hf, Woh = _ln_final(
        h2s, _unsplit_half(agg2in2), _unsplit_half(agg2out2),
        row(g2), row(be2), FT, GT, HT, currency_table, format_table)

    SH, DH = _edge_gather(hf, src, dst)

    sm = jnp.concatenate([
        edge_numeric.astype(f32),
        jnp.ones((E, 1), f32),
        jnp.zeros((E, 1), f32),
        edge_sent_currency.astype(f32).reshape(E, 1),
        edge_recv_currency.astype(f32).reshape(E, 1),
        edge_payment_format.astype(f32).reshape(E, 1),
        jnp.zeros((E, 5), f32),
    ], axis=1)
    Wsm = jnp.concatenate([Et, bm1.reshape(1, D), jnp.zeros((9, D), f32)], axis=0)
    wm3 = Wm3.reshape(1, -1)
    out8 = _edge_mlp(SH, DH, sm,
                     A.T, B.T, C.T, Dm.T, Woh, Wsm,
                     Wm2.T, bm2.reshape(1, -1), wm3, bm3.reshape(1, 1))
    return out8[:, 0]


# speedup vs baseline: 3.3331x; 1.0017x over previous
import functools

import jax
import jax.numpy as jnp
from jax import lax
from jax.experimental import pallas as pl
from jax.experimental.pallas import tpu as pltpu
from jax.experimental.pallas import tpu_sc as plsc

N = 10000
E = 160000
D = 256
HALF = 128
NC, NS = 2, 16
NW = NC * NS
KE = 128
NCHUNK = E // KE
NB_PAD = 10240

_bf = lambda v: v.astype(jnp.bfloat16)


def _vsm():
    return plsc.VectorSubcoreMesh(
        core_axis_name="c", subcore_axis_name="s", num_cores=NC, num_subcores=NS)


def _bank_gather(table, idx_pad):
    nchunk = NB_PAD // KE

    @functools.partial(
        pl.kernel,
        out_type=jax.ShapeDtypeStruct((NB_PAD, HALF), jnp.float32),
        mesh=_vsm(),
        scratch_types=[
            pltpu.VMEM((KE,), jnp.int32),
            pltpu.VMEM((KE, HALF), jnp.float32),
            pltpu.SemaphoreType.DMA,
        ],
    )
    def k(table_hbm, idx_hbm, out_hbm, idx_v, rows_v, sem):
        cid = lax.axis_index("c")
        sid = lax.axis_index("s")
        wid = sid * NC + cid

        def body(j, _):
            kk = wid + NW * j

            @pl.when(kk < nchunk)
            def _():
                base = kk * KE
                pltpu.sync_copy(idx_hbm.at[pl.ds(base, KE)], idx_v)
                pltpu.async_copy(table_hbm.at[idx_v], rows_v, sem).wait()
                pltpu.sync_copy(rows_v, out_hbm.at[pl.ds(base, KE)])
            return 0

        lax.fori_loop(0, (nchunk + NW - 1) // NW, body, 0)

    return k(table, idx_pad)


def _seg_body_helpers(cid, sid, gbufs, sbufs, rbufs, zbuf_v, acc_sh, sems):
    gidx0_v, gidx1_v = gbufs
    sidx0_v, sidx1_v = sbufs
    rows0_v, rows1_v = rbufs
    sem0, sem1 = sems
    nzero = N // 16

    def zero_fill():
        for r in range(16):
            for c8 in range(HALF // 16):
                zbuf_v[r, pl.ds(c8 * 16, 16)] = jnp.zeros((16,), jnp.float32)

    def zero_acc():
        def zb(j, _):
            kk = sid + NS * j

            @pl.when(kk < nzero)
            def _():
                pltpu.sync_copy(zbuf_v, acc_sh.at[pl.ds(kk * 16, 16)])
            return 0

        lax.fori_loop(0, (nzero + NS - 1) // NS, zb, 0)

    def accumulate(tab_hbm, gat_hbm, sct_hbm, interleave):
        def start(j, gidx_v, sidx_v, rows_v, sem):
            kk = sid + NS * j

            @pl.when(kk < NCHUNK)
            def _():
                base = kk * KE
                pltpu.sync_copy(gat_hbm.at[pl.ds(base, KE)], gidx_v)
                if interleave:
                    for c16 in range(KE // 16):
                        sl = pl.ds(c16 * 16, 16)
                        gidx_v[sl] = gidx_v[sl] * 2 + cid
                pltpu.sync_copy(sct_hbm.at[pl.ds(base, KE)], sidx_v)
                pltpu.async_copy(tab_hbm.at[gidx_v], rows_v, sem)

        def finish(j, gidx_v, sidx_v, rows_v, sem):
            kk = sid + NS * j

            @pl.when(kk < NCHUNK)
            def _():
                pltpu.make_async_copy(tab_hbm.at[gidx_v], rows_v, sem).wait()
                pltpu.sync_copy(rows_v, acc_sh.at[sidx_v], add=True)

        start(0, gidx0_v, sidx0_v, rows0_v, sem0)

        def outer(jj, _):
            j0 = 2 * jj
            start(j0 + 1, gidx1_v, sidx1_v, rows1_v, sem1)
            finish(j0, gidx0_v, sidx0_v, rows0_v, sem0)
            start(j0 + 2, gidx0_v, sidx0_v, rows0_v, sem0)
            finish(j0 + 1, gidx1_v, sidx1_v, rows1_v, sem1)
            return 0

        lax.fori_loop(0, 40, outer, 0)

    def writeout(out_hbm, row_off):
        def wb(j, _):
            kk = sid + NS * j

            @pl.when(kk < nzero)
            def _():
                pltpu.sync_copy(acc_sh.at[pl.ds(kk * 16, 16)],
                                out_hbm.at[pl.ds(row_off + kk * 16, 16)])
            return 0

        lax.fori_loop(0, (nzero + NS - 1) // NS, wb, 0)

    return zero_fill, zero_acc, accumulate, writeout


_SEG_SCRATCH = [
    pltpu.VMEM((KE,), jnp.int32),
    pltpu.VMEM((KE,), jnp.int32),
    pltpu.VMEM((KE,), jnp.int32),
    pltpu.VMEM((KE,), jnp.int32),
    pltpu.VMEM((KE, HALF), jnp.float32),
    pltpu.VMEM((KE, HALF), jnp.float32),
    pltpu.VMEM((16, HALF), jnp.float32),
    pltpu.VMEM_SHARED((N, HALF), jnp.float32),
    pltpu.SemaphoreType.DMA,
    pltpu.SemaphoreType.DMA,
]


def _segsum_l1(nf2, bepad, src, dst):
    sds2 = jax.ShapeDtypeStruct((NC * N, HALF), jnp.float32)

    @functools.partial(
        pl.kernel,
        out_type=(sds2, sds2, sds2),
        mesh=_vsm(),
        scratch_types=_SEG_SCRATCH,
    )
    def k(nf_hbm, be_hbm, src_hbm, dst_hbm, ain_hbm, aout_hbm, abk_hbm,
          gidx0_v, gidx1_v, sidx0_v, sidx1_v, rows0_v, rows1_v,
          zbuf_v, acc_sh, sem0, sem1):
        cid = lax.axis_index("c")
        sid = lax.axis_index("s")
        row_off = cid * N
        zero_fill, zero_acc, accumulate, writeout = _seg_body_helpers(
            cid, sid, (gidx0_v, gidx1_v), (sidx0_v, sidx1_v),
            (rows0_v, rows1_v), zbuf_v, acc_sh, (sem0, sem1))

        zero_fill()
        zero_acc()
        plsc.subcore_barrier()
        accumulate(nf_hbm, src_hbm, dst_hbm, True)
        plsc.subcore_barrier()
        writeout(ain_hbm, row_off)
        plsc.subcore_barrier()
        zero_acc()
        plsc.subcore_barrier()
        accumulate(nf_hbm, dst_hbm, src_hbm, True)
        plsc.subcore_barrier()
        writeout(aout_hbm, row_off)
        plsc.subcore_barrier()
        zero_acc()
        plsc.subcore_barrier()

        @pl.when(cid == 0)
        def _():
            accumulate(be_hbm, src_hbm, dst_hbm, False)

        @pl.when(cid == 1)
        def _():
            accumulate(be_hbm, dst_hbm, src_hbm, False)

        plsc.subcore_barrier()
        writeout(abk_hbm, row_off)

    return k(nf2, bepad, src, dst)


def _segsum_pair(tab2, src, dst):
    sds2 = jax.ShapeDtypeStruct((NC * N, HALF), jnp.float32)

    @functools.partial(
        pl.kernel,
        out_type=(sds2, sds2),
        mesh=_vsm(),
        scratch_types=_SEG_SCRATCH,
    )
    def k(tab_hbm, src_hbm, dst_hbm, ain_hbm, aout_hbm,
          gidx0_v, gidx1_v, sidx0_v, sidx1_v, rows0_v, rows1_v,
          zbuf_v, acc_sh, sem0, sem1):
        cid = lax.axis_index("c")
        sid = lax.axis_index("s")
        row_off = cid * N
        zero_fill, zero_acc, accumulate, writeout = _seg_body_helpers(
            cid, sid, (gidx0_v, gidx1_v), (sidx0_v, sidx1_v),
            (rows0_v, rows1_v), zbuf_v, acc_sh, (sem0, sem1))

        zero_fill()
        zero_acc()
        plsc.subcore_barrier()
        accumulate(tab_hbm, src_hbm, dst_hbm, True)
        plsc.subcore_barrier()
        writeout(ain_hbm, row_off)
        plsc.subcore_barrier()
        zero_acc()
        plsc.subcore_barrier()
        accumulate(tab_hbm, dst_hbm, src_hbm, True)
        plsc.subcore_barrier()
        writeout(aout_hbm, row_off)

    return k(tab2, src, dst)


def _edge_gather(hf, src, dst):
    out_sds = jax.ShapeDtypeStruct((E, D), jnp.float32)

    @functools.partial(
        pl.kernel,
        out_type=(out_sds, out_sds),
        mesh=_vsm(),
        scratch_types=[
            pltpu.VMEM((KE,), jnp.int32),
            pltpu.VMEM((KE,), jnp.int32),
            pltpu.VMEM((KE, D), jnp.float32),
            pltpu.VMEM((KE, D), jnp.float32),
            pltpu.SemaphoreType.DMA,
            pltpu.SemaphoreType.DMA,
        ],
    )
    def k(hf_hbm, src_hbm, dst_hbm, sh_hbm, dh_hbm,
          idx0_v, idx1_v, rows0_v, rows1_v, sem0, sem1):
        cid = lax.axis_index("c")
        sid = lax.axis_index("s")
        wid = sid * NC + cid

        def start(eidx_hbm, j, idx_v, rows_v, sem):
            kk = wid + NW * j

            @pl.when(kk < NCHUNK)
            def _():
                pltpu.sync_copy(eidx_hbm.at[pl.ds(kk * KE, KE)], idx_v)
                pltpu.async_copy(hf_hbm.at[idx_v], rows_v, sem)

        def finish(out_hbm, j, idx_v, rows_v, sem):
            kk = wid + NW * j

            @pl.when(kk < NCHUNK)
            def _():
                pltpu.make_async_copy(hf_hbm.at[idx_v], rows_v, sem).wait()
                pltpu.sync_copy(rows_v, out_hbm.at[pl.ds(kk * KE, KE)])

        start(src_hbm, 0, idx0_v, rows0_v, sem0)

        def body(j, _):
            start(dst_hbm, j, idx1_v, rows1_v, sem1)
            finish(sh_hbm, j, idx0_v, rows0_v, sem0)
            start(src_hbm, j + 1, idx0_v, rows0_v, sem0)
            finish(dh_hbm, j, idx1_v, rows1_v, sem1)
            return 0

        lax.fori_loop(0, (NCHUNK + NW - 1) // NW, body, 0)

    return k(hf, src, dst)


BN = 400


def _combine1(nf, bepad, ain2, abk2, aout2, WsaT, WsbT, WiaT, WibT,
              WoaT, WobT, b1, g, b):
    def body(nf_r, be_r, ai0_r, ai1_r, abi_r, ao0_r, ao1_r, abo_r,
             g_r, b_r, wsa, wsb, wia, wib, woa, wob, b1_r, h_r):
        dot = lambda x, w: jnp.dot(_bf(x), _bf(w[...]),
                                   preferred_element_type=jnp.float32)
        ai = jnp.concatenate([ai0_r[...], ai1_r[...]], axis=1)
        ao = jnp.concatenate([ao0_r[...], ao1_r[...]], axis=1)
        pre = (dot(nf_r[...], wsa) + dot(be_r[...][:, :16], wsb) + b1_r[...]
               + dot(ai, wia) + dot(abi_r[...][:, :16], wib)
               + dot(ao, woa) + dot(abo_r[...][:, :16], wob))
        m = jnp.mean(pre, axis=1, keepdims=True)
        v = jnp.mean((pre - m) * (pre - m), axis=1, keepdims=True)
        h = (pre - m) * lax.rsqrt(v + 1e-5) * g_r[...] + b_r[...]
        h_r[...] = jnp.maximum(h, 0.0)

    full = lambda s: pl.BlockSpec(s, lambda i: (0,) * len(s))
    blk = pl.BlockSpec((BN, D), lambda i: (i, 0))
    bh = pl.BlockSpec((BN, HALF), lambda i: (i, 0))
    h0 = pl.BlockSpec((BN, HALF), lambda i: (i, 0))
    h1 = pl.BlockSpec((BN, HALF), lambda i: (N // BN + i, 0))
    return pl.pallas_call(
        body,
        grid=(N // BN,),
        in_specs=[blk, bh, h0, h1, h0, h0, h1, h1,
                  full((1, D)), full((1, D)),
                  full((D, D)), full((16, D)), full((D, D)), full((16, D)),
                  full((D, D)), full((16, D)), full((1, D))],
        out_specs=[blk],
        out_shape=[jax.ShapeDtypeStruct((N, D), jnp.float32)],
    )(nf, bepad, ain2, ain2, abk2, aout2, aout2, abk2,
      g, b, WsaT, WsbT, WiaT, WibT, WoaT, WobT, b1)[0]


def _combine2(h, ain2, aout2, W2sT, W2iT, W2oT, b2, g, b, FT, GT, HT, ct, ft):
    def body(h_r, ai0_r, ai1_r, ao0_r, ao1_r, g_r, b_r,
             ws, wi, wo, b2_r, ftab_r, gtab_r, htab_r, ct_r, ftm_r,
             hf_r, woh_r):
        dot = lambda x, w: jnp.dot(_bf(x), _bf(w[...]),
                                   preferred_element_type=jnp.float32)
        ai = jnp.concatenate([ai0_r[...], ai1_r[...]], axis=1)
        ao = jnp.concatenate([ao0_r[...], ao1_r[...]], axis=1)
        pre = (dot(h_r[...], ws) + b2_r[...] + dot(ai, wi) + dot(ao, wo))
        m = jnp.mean(pre, axis=1, keepdims=True)
        v = jnp.mean((pre - m) * (pre - m), axis=1, keepdims=True)
        hh = (pre - m) * lax.rsqrt(v + 1e-5) * g_r[...] + b_r[...]
        hf_r[...] = jnp.maximum(hh, 0.0)

        @pl.when(pl.program_id(0) == 0)
        def _():
            dotc = lambda x, w: jnp.dot(_bf(x), _bf(w),
                                        preferred_element_type=jnp.float32)
            woh_r[...] = jnp.zeros((128, D), jnp.float32)
            woh_r[0:32, :] = dotc(ct_r[...], ftab_r[...])
            woh_r[32:64, :] = dotc(ct_r[...], gtab_r[...])
            woh_r[64:72, :] = dotc(ftm_r[...], htab_r[...])

    full = lambda s: pl.BlockSpec(s, lambda i: (0,) * len(s))
    blk = pl.BlockSpec((BN, D), lambda i: (i, 0))
    h0 = pl.BlockSpec((BN, HALF), lambda i: (i, 0))
    h1 = pl.BlockSpec((BN, HALF), lambda i: (N // BN + i, 0))
    return pl.pallas_call(
        body,
        grid=(N // BN,),
        in_specs=[blk, h0, h1, h0, h1, full((1, D)), full((1, D)),
                  full((D, D)), full((D, D)), full((D, D)), full((1, D)),
                  full((8, D)), full((8, D)), full((8, D)),
                  full((32, 8)), full((8, 8))],
        out_specs=[blk, full((128, D))],
        out_shape=[jax.ShapeDtypeStruct((N, D), jnp.float32),
                   jax.ShapeDtypeStruct((128, D), jnp.float32)],
    )(h, ain2, ain2, aout2, aout2, g, b, W2sT, W2iT, W2oT, b2,
      FT, GT, HT, ct, ft)


BE = 1600


def _edge_mlp(SH, DH, sm, Wcat5, Wsm, Wm2T, bm2, wm3, bm3):
    def body(sh_r, dh_r, sm_r, wcat, wsm, w2, b2, w3, b3, out_r):
        sh = sh_r[...]
        dh = dh_r[...]
        sm_ = sm_r[...]
        dot = lambda x, w: jnp.dot(_bf(x), _bf(w[...]),
                                   preferred_element_type=jnp.float32)
        lanes = lax.broadcasted_iota(jnp.int32, (BE, 128), 1)
        cs = sm_[:, 8:9].astype(jnp.int32)
        cr = sm_[:, 9:10].astype(jnp.int32)
        pf = sm_[:, 10:11].astype(jnp.int32)
        oh = ((lanes == cs) | (lanes == 32 + cr) |
              (lanes == 64 + pf)).astype(jnp.float32)
        cat = jnp.concatenate([sh, dh, jnp.abs(sh - dh), sh * dh, oh], axis=1)
        z1 = jnp.maximum(dot(cat, wcat) + dot(sm_, wsm), 0.0)
        z2 = jnp.maximum(dot(z1, w2) + b2[...], 0.0)
        z2r = _bf(z2).astype(jnp.float32)
        w3r = _bf(w3[...]).astype(jnp.float32)
        o = jnp.sum(z2r * w3r, axis=1, keepdims=True) + b3[...]
        out_r[...] = jnp.broadcast_to(o, (BE, 8))

    full = lambda s: pl.BlockSpec(s, lambda i: (0,) * len(s))
    blk = pl.BlockSpec((BE, D), lambda i: (i, 0))
    return pl.pallas_call(
        body,
        grid=(E // BE,),
        in_specs=[blk, blk, pl.BlockSpec((BE, 16), lambda i: (i, 0)),
                  full((4 * D + 128, D)), full((16, D)),
                  full((D, 128)), full((1, 128)), full((1, 128)), full((1, 1))],
        out_specs=[pl.BlockSpec((BE, 8), lambda i: (i, 0))],
        out_shape=[jax.ShapeDtypeStruct((E, 8), jnp.float32)],
    )(SH, DH, sm, Wcat5, Wsm, Wm2T, bm2, wm3, bm3)[0]


def kernel(node_features, edge_numeric, node_bank_ids, edge_src, edge_dst,
           edge_sent_currency, edge_recv_currency, edge_payment_format,
           bank_table, currency_table, format_table,
           W1_self, b1_self, W1_in, W1_out, g1, be1,
           W2_self, b2_self, W2_in, W2_out, g2, be2,
           Wm1, bm1, Wm2, bm2, Wm3, bm3):
    f32 = jnp.float32
    src = edge_src.astype(jnp.int32)
    dst = edge_dst.astype(jnp.int32)
    row = lambda v: v.reshape(1, -1)

    bids_pad = jnp.pad(node_bank_ids.astype(jnp.int32), (0, NB_PAD - N))
    bank_pad = jnp.pad(bank_table, ((0, 0), (0, HALF - 16)))
    bepad = _bank_gather(bank_pad, bids_pad)

    ain2, aout2, abk2 = _segsum_l1(
        node_features.reshape(2 * N, HALF), bepad, src, dst)

    h = _combine1(node_features, bepad, ain2, abk2, aout2,
                  W1_self[:, :D].T, W1_self[:, D:].T,
                  W1_in[:, :D].T, W1_in[:, D:].T,
                  W1_out[:, :D].T, W1_out[:, D:].T,
                  row(b1_self), row(g1), row(be1))

    h2 = h.reshape(2 * N, HALF)
    a2in2, a2out2 = _segsum_pair(h2, src, dst)

    A = Wm1[:, 0:256]
    B = Wm1[:, 256:512]
    C = Wm1[:, 512:768]
    Dm = Wm1[:, 768:1024]
    Et = Wm1[:, 1024:1030].T
    FT = Wm1[:, 1030:1038].T
    GT = Wm1[:, 1038:1046].T
    HT = Wm1[:, 1046:1054].T
    hf, Woh = _combine2(h, a2in2, a2out2,
                        W2_self.T, W2_in.T, W2_out.T,
                        row(b2_self), row(g2), row(be2),
                        FT, GT, HT, currency_table, format_table)

    SH, DH = _edge_gather(hf, src, dst)

    sm = jnp.concatenate([
        edge_numeric.astype(f32),
        jnp.ones((E, 1), f32),
        jnp.zeros((E, 1), f32),
        edge_sent_currency.astype(f32).reshape(E, 1),
        edge_recv_currency.astype(f32).reshape(E, 1),
        edge_payment_format.astype(f32).reshape(E, 1),
        jnp.zeros((E, 5), f32),
    ], axis=1)
    Wsm = jnp.concatenate([Et, bm1.reshape(1, D), jnp.zeros((9, D), f32)],
                          axis=0)
    Wcat5 = jnp.concatenate([A.T, B.T, C.T, Dm.T, Woh], axis=0)
    wm3 = Wm3.reshape(1, -1)
    out8 = _edge_mlp(SH, DH, sm, Wcat5, Wsm,
                     Wm2.T, bm2.reshape(1, -1), wm3, bm3.reshape(1, 1))
    return out8[:, 0]


# speedup vs baseline: 3.4492x; 1.0348x over previous
import functools

import jax
import jax.numpy as jnp
from jax import lax
from jax.experimental import pallas as pl
from jax.experimental.pallas import tpu as pltpu
from jax.experimental.pallas import tpu_sc as plsc

N = 10000
E = 160000
D = 256
HALF = 128
NC, NS = 2, 16
NW = NC * NS
KE = 128
NCHUNK = E // KE
NB_PAD = 10240

_bf = lambda v: v.astype(jnp.bfloat16)


def _vsm():
    return plsc.VectorSubcoreMesh(
        core_axis_name="c", subcore_axis_name="s", num_cores=NC, num_subcores=NS)


def _bank_gather(table, idx_pad):
    nchunk = NB_PAD // KE

    @functools.partial(
        pl.kernel,
        out_type=jax.ShapeDtypeStruct((NB_PAD, HALF), jnp.float32),
        mesh=_vsm(),
        scratch_types=[
            pltpu.VMEM((KE,), jnp.int32),
            pltpu.VMEM((KE, HALF), jnp.float32),
            pltpu.SemaphoreType.DMA,
        ],
    )
    def k(table_hbm, idx_hbm, out_hbm, idx_v, rows_v, sem):
        cid = lax.axis_index("c")
        sid = lax.axis_index("s")
        wid = sid * NC + cid

        def body(j, _):
            kk = wid + NW * j

            @pl.when(kk < nchunk)
            def _():
                base = kk * KE
                pltpu.sync_copy(idx_hbm.at[pl.ds(base, KE)], idx_v)
                pltpu.async_copy(table_hbm.at[idx_v], rows_v, sem).wait()
                pltpu.sync_copy(rows_v, out_hbm.at[pl.ds(base, KE)])
            return 0

        lax.fori_loop(0, (nchunk + NW - 1) // NW, body, 0)

    return k(table, idx_pad)


def _seg_body_helpers(cid, sid, gbufs, sbufs, rbufs, zbuf_v, acc_sh, sems):
    gidx0_v, gidx1_v = gbufs
    sidx0_v, sidx1_v = sbufs
    rows0_v, rows1_v = rbufs
    sem0, sem1 = sems
    nzero = N // 16

    def zero_fill():
        for r in range(16):
            for c8 in range(HALF // 16):
                zbuf_v[r, pl.ds(c8 * 16, 16)] = jnp.zeros((16,), jnp.float32)

    def zero_acc():
        def zb(j, _):
            kk = sid + NS * j

            @pl.when(kk < nzero)
            def _():
                pltpu.sync_copy(zbuf_v, acc_sh.at[pl.ds(kk * 16, 16)])
            return 0

        lax.fori_loop(0, (nzero + NS - 1) // NS, zb, 0)

    def accumulate(tab_hbm, gat_hbm, sct_hbm, interleave):
        def start(j, gidx_v, sidx_v, rows_v, sem):
            kk = sid + NS * j

            @pl.when(kk < NCHUNK)
            def _():
                base = kk * KE
                pltpu.sync_copy(gat_hbm.at[pl.ds(base, KE)], gidx_v)
                if interleave:
                    for c16 in range(KE // 16):
                        sl = pl.ds(c16 * 16, 16)
                        gidx_v[sl] = gidx_v[sl] * 2 + cid
                pltpu.sync_copy(sct_hbm.at[pl.ds(base, KE)], sidx_v)
                pltpu.async_copy(tab_hbm.at[gidx_v], rows_v, sem)

        def finish(j, gidx_v, sidx_v, rows_v, sem):
            kk = sid + NS * j

            @pl.when(kk < NCHUNK)
            def _():
                pltpu.make_async_copy(tab_hbm.at[gidx_v], rows_v, sem).wait()
                pltpu.sync_copy(rows_v, acc_sh.at[sidx_v], add=True)

        start(0, gidx0_v, sidx0_v, rows0_v, sem0)

        def outer(jj, _):
            j0 = 2 * jj
            start(j0 + 1, gidx1_v, sidx1_v, rows1_v, sem1)
            finish(j0, gidx0_v, sidx0_v, rows0_v, sem0)
            start(j0 + 2, gidx0_v, sidx0_v, rows0_v, sem0)
            finish(j0 + 1, gidx1_v, sidx1_v, rows1_v, sem1)
            return 0

        lax.fori_loop(0, 40, outer, 0)

    def writeout(out_hbm, row_off):
        def wb(j, _):
            kk = sid + NS * j

            @pl.when(kk < nzero)
            def _():
                pltpu.sync_copy(acc_sh.at[pl.ds(kk * 16, 16)],
                                out_hbm.at[pl.ds(row_off + kk * 16, 16)])
            return 0

        lax.fori_loop(0, (nzero + NS - 1) // NS, wb, 0)

    return zero_fill, zero_acc, accumulate, writeout


_SEG_SCRATCH = [
    pltpu.VMEM((KE,), jnp.int32),
    pltpu.VMEM((KE,), jnp.int32),
    pltpu.VMEM((KE,), jnp.int32),
    pltpu.VMEM((KE,), jnp.int32),
    pltpu.VMEM((KE, HALF), jnp.float32),
    pltpu.VMEM((KE, HALF), jnp.float32),
    pltpu.VMEM((16, HALF), jnp.float32),
    pltpu.VMEM_SHARED((N, HALF), jnp.float32),
    pltpu.SemaphoreType.DMA,
    pltpu.SemaphoreType.DMA,
]


def _segsum_l1(nf2, bepad, src, dst):
    sds2 = jax.ShapeDtypeStruct((NC * N, HALF), jnp.float32)

    @functools.partial(
        pl.kernel,
        out_type=(sds2, sds2, sds2),
        mesh=_vsm(),
        scratch_types=_SEG_SCRATCH,
    )
    def k(nf_hbm, be_hbm, src_hbm, dst_hbm, ain_hbm, aout_hbm, abk_hbm,
          gidx0_v, gidx1_v, sidx0_v, sidx1_v, rows0_v, rows1_v,
          zbuf_v, acc_sh, sem0, sem1):
        cid = lax.axis_index("c")
        sid = lax.axis_index("s")
        row_off = cid * N
        zero_fill, zero_acc, accumulate, writeout = _seg_body_helpers(
            cid, sid, (gidx0_v, gidx1_v), (sidx0_v, sidx1_v),
            (rows0_v, rows1_v), zbuf_v, acc_sh, (sem0, sem1))

        zero_fill()
        zero_acc()
        plsc.subcore_barrier()
        accumulate(nf_hbm, src_hbm, dst_hbm, True)
        plsc.subcore_barrier()
        writeout(ain_hbm, row_off)
        plsc.subcore_barrier()
        zero_acc()
        plsc.subcore_barrier()
        accumulate(nf_hbm, dst_hbm, src_hbm, True)
        plsc.subcore_barrier()
        writeout(aout_hbm, row_off)
        plsc.subcore_barrier()
        zero_acc()
        plsc.subcore_barrier()

        @pl.when(cid == 0)
        def _():
            accumulate(be_hbm, src_hbm, dst_hbm, False)

        @pl.when(cid == 1)
        def _():
            accumulate(be_hbm, dst_hbm, src_hbm, False)

        plsc.subcore_barrier()
        writeout(abk_hbm, row_off)

    return k(nf2, bepad, src, dst)


def _segsum_pair(tab2, src, dst):
    sds2 = jax.ShapeDtypeStruct((NC * N, HALF), jnp.float32)

    @functools.partial(
        pl.kernel,
        out_type=(sds2, sds2),
        mesh=_vsm(),
        scratch_types=_SEG_SCRATCH,
    )
    def k(tab_hbm, src_hbm, dst_hbm, ain_hbm, aout_hbm,
          gidx0_v, gidx1_v, sidx0_v, sidx1_v, rows0_v, rows1_v,
          zbuf_v, acc_sh, sem0, sem1):
        cid = lax.axis_index("c")
        sid = lax.axis_index("s")
        row_off = cid * N
        zero_fill, zero_acc, accumulate, writeout = _seg_body_helpers(
            cid, sid, (gidx0_v, gidx1_v), (sidx0_v, sidx1_v),
            (rows0_v, rows1_v), zbuf_v, acc_sh, (sem0, sem1))

        zero_fill()
        zero_acc()
        plsc.subcore_barrier()
        accumulate(tab_hbm, src_hbm, dst_hbm, True)
        plsc.subcore_barrier()
        writeout(ain_hbm, row_off)
        plsc.subcore_barrier()
        zero_acc()
        plsc.subcore_barrier()
        accumulate(tab_hbm, dst_hbm, src_hbm, True)
        plsc.subcore_barrier()
        writeout(aout_hbm, row_off)

    return k(tab2, src, dst)


def _edge_gather(hf, src, dst, ne):
    nchunk = ne // KE
    out_sds = jax.ShapeDtypeStruct((ne, D), jnp.float32)

    @functools.partial(
        pl.kernel,
        out_type=(out_sds, out_sds),
        mesh=_vsm(),
        scratch_types=[
            pltpu.VMEM((KE,), jnp.int32),
            pltpu.VMEM((KE,), jnp.int32),
            pltpu.VMEM((KE, D), jnp.float32),
            pltpu.VMEM((KE, D), jnp.float32),
            pltpu.SemaphoreType.DMA,
            pltpu.SemaphoreType.DMA,
        ],
    )
    def k(hf_hbm, src_hbm, dst_hbm, sh_hbm, dh_hbm,
          idx0_v, idx1_v, rows0_v, rows1_v, sem0, sem1):
        cid = lax.axis_index("c")
        sid = lax.axis_index("s")
        wid = sid * NC + cid

        def start(eidx_hbm, j, idx_v, rows_v, sem):
            kk = wid + NW * j

            @pl.when(kk < nchunk)
            def _():
                pltpu.sync_copy(eidx_hbm.at[pl.ds(kk * KE, KE)], idx_v)
                pltpu.async_copy(hf_hbm.at[idx_v], rows_v, sem)

        def finish(out_hbm, j, idx_v, rows_v, sem):
            kk = wid + NW * j

            @pl.when(kk < nchunk)
            def _():
                pltpu.make_async_copy(hf_hbm.at[idx_v], rows_v, sem).wait()
                pltpu.sync_copy(rows_v, out_hbm.at[pl.ds(kk * KE, KE)])

        start(src_hbm, 0, idx0_v, rows0_v, sem0)

        def body(j, _):
            start(dst_hbm, j, idx1_v, rows1_v, sem1)
            finish(sh_hbm, j, idx0_v, rows0_v, sem0)
            start(src_hbm, j + 1, idx0_v, rows0_v, sem0)
            finish(dh_hbm, j, idx1_v, rows1_v, sem1)
            return 0

        lax.fori_loop(0, (nchunk + NW - 1) // NW, body, 0)

    return k(hf, src, dst)


BN = 400


def _combine1(nf, bepad, ain2, abk2, aout2, WsaT, WsbT, WiaT, WibT,
              WoaT, WobT, b1, g, b):
    def body(nf_r, be_r, ai0_r, ai1_r, abi_r, ao0_r, ao1_r, abo_r,
             g_r, b_r, wsa, wsb, wia, wib, woa, wob, b1_r, h_r):
        dot = lambda x, w: jnp.dot(_bf(x), _bf(w[...]),
                                   preferred_element_type=jnp.float32)
        ai = jnp.concatenate([ai0_r[...], ai1_r[...]], axis=1)
        ao = jnp.concatenate([ao0_r[...], ao1_r[...]], axis=1)
        pre = (dot(nf_r[...], wsa) + dot(be_r[...][:, :16], wsb) + b1_r[...]
               + dot(ai, wia) + dot(abi_r[...][:, :16], wib)
               + dot(ao, woa) + dot(abo_r[...][:, :16], wob))
        m = jnp.mean(pre, axis=1, keepdims=True)
        v = jnp.mean((pre - m) * (pre - m), axis=1, keepdims=True)
        h = (pre - m) * lax.rsqrt(v + 1e-5) * g_r[...] + b_r[...]
        h_r[...] = jnp.maximum(h, 0.0)

    full = lambda s: pl.BlockSpec(s, lambda i: (0,) * len(s))
    blk = pl.BlockSpec((BN, D), lambda i: (i, 0))
    bh = pl.BlockSpec((BN, HALF), lambda i: (i, 0))
    h0 = pl.BlockSpec((BN, HALF), lambda i: (i, 0))
    h1 = pl.BlockSpec((BN, HALF), lambda i: (N // BN + i, 0))
    return pl.pallas_call(
        body,
        grid=(N // BN,),
        in_specs=[blk, bh, h0, h1, h0, h0, h1, h1,
                  full((1, D)), full((1, D)),
                  full((D, D)), full((16, D)), full((D, D)), full((16, D)),
                  full((D, D)), full((16, D)), full((1, D))],
        out_specs=[blk],
        out_shape=[jax.ShapeDtypeStruct((N, D), jnp.float32)],
    )(nf, bepad, ain2, ain2, abk2, aout2, aout2, abk2,
      g, b, WsaT, WsbT, WiaT, WibT, WoaT, WobT, b1)[0]


def _combine2(h, ain2, aout2, W2sT, W2iT, W2oT, b2, g, b, FT, GT, HT, ct, ft):
    def body(h_r, ai0_r, ai1_r, ao0_r, ao1_r, g_r, b_r,
             ws, wi, wo, b2_r, ftab_r, gtab_r, htab_r, ct_r, ftm_r,
             hf_r, woh_r):
        dot = lambda x, w: jnp.dot(_bf(x), _bf(w[...]),
                                   preferred_element_type=jnp.float32)
        ai = jnp.concatenate([ai0_r[...], ai1_r[...]], axis=1)
        ao = jnp.concatenate([ao0_r[...], ao1_r[...]], axis=1)
        pre = (dot(h_r[...], ws) + b2_r[...] + dot(ai, wi) + dot(ao, wo))
        m = jnp.mean(pre, axis=1, keepdims=True)
        v = jnp.mean((pre - m) * (pre - m), axis=1, keepdims=True)
        hh = (pre - m) * lax.rsqrt(v + 1e-5) * g_r[...] + b_r[...]
        hf_r[...] = jnp.maximum(hh, 0.0)

        @pl.when(pl.program_id(0) == 0)
        def _():
            dotc = lambda x, w: jnp.dot(_bf(x), _bf(w),
                                        preferred_element_type=jnp.float32)
            woh_r[...] = jnp.zeros((128, D), jnp.float32)
            woh_r[0:32, :] = dotc(ct_r[...], ftab_r[...])
            woh_r[32:64, :] = dotc(ct_r[...], gtab_r[...])
            woh_r[64:72, :] = dotc(ftm_r[...], htab_r[...])

    full = lambda s: pl.BlockSpec(s, lambda i: (0,) * len(s))
    blk = pl.BlockSpec((BN, D), lambda i: (i, 0))
    h0 = pl.BlockSpec((BN, HALF), lambda i: (i, 0))
    h1 = pl.BlockSpec((BN, HALF), lambda i: (N // BN + i, 0))
    return pl.pallas_call(
        body,
        grid=(N // BN,),
        in_specs=[blk, h0, h1, h0, h1, full((1, D)), full((1, D)),
                  full((D, D)), full((D, D)), full((D, D)), full((1, D)),
                  full((8, D)), full((8, D)), full((8, D)),
                  full((32, 8)), full((8, 8))],
        out_specs=[blk, full((128, D))],
        out_shape=[jax.ShapeDtypeStruct((N, D), jnp.float32),
                   jax.ShapeDtypeStruct((128, D), jnp.float32)],
    )(h, ain2, ain2, aout2, aout2, g, b, W2sT, W2iT, W2oT, b2,
      FT, GT, HT, ct, ft)


BE = 1600


def _edge_mlp(SH, DH, sm, Wcat5, Wsm, Wm2T, bm2, wm3, bm3, ne):
    def body(sh_r, dh_r, sm_r, wcat, wsm, w2, b2, w3, b3, out_r):
        sh = sh_r[...]
        dh = dh_r[...]
        sm_ = sm_r[...]
        dot = lambda x, w: jnp.dot(_bf(x), _bf(w[...]),
                                   preferred_element_type=jnp.float32)
        lanes = lax.broadcasted_iota(jnp.int32, (BE, 128), 1)
        cs = sm_[:, 8:9].astype(jnp.int32)
        cr = sm_[:, 9:10].astype(jnp.int32)
        pf = sm_[:, 10:11].astype(jnp.int32)
        oh = ((lanes == cs) | (lanes == 32 + cr) |
              (lanes == 64 + pf)).astype(jnp.float32)
        cat = jnp.concatenate([sh, dh, jnp.abs(sh - dh), sh * dh, oh], axis=1)
        z1 = jnp.maximum(dot(cat, wcat) + dot(sm_, wsm), 0.0)
        z2 = jnp.maximum(dot(z1, w2) + b2[...], 0.0)
        z2r = _bf(z2).astype(jnp.float32)
        w3r = _bf(w3[...]).astype(jnp.float32)
        o = jnp.sum(z2r * w3r, axis=1, keepdims=True) + b3[...]
        out_r[...] = jnp.broadcast_to(o, (BE, 8))

    full = lambda s: pl.BlockSpec(s, lambda i: (0,) * len(s))
    blk = pl.BlockSpec((BE, D), lambda i: (i, 0))
    return pl.pallas_call(
        body,
        grid=(ne // BE,),
        in_specs=[blk, blk, pl.BlockSpec((BE, 16), lambda i: (i, 0)),
                  full((4 * D + 128, D)), full((16, D)),
                  full((D, 128)), full((1, 128)), full((1, 128)), full((1, 1))],
        out_specs=[pl.BlockSpec((BE, 8), lambda i: (i, 0))],
        out_shape=[jax.ShapeDtypeStruct((ne, 8), jnp.float32)],
    )(SH, DH, sm, Wcat5, Wsm, Wm2T, bm2, wm3, bm3)[0]


def kernel(node_features, edge_numeric, node_bank_ids, edge_src, edge_dst,
           edge_sent_currency, edge_recv_currency, edge_payment_format,
           bank_table, currency_table, format_table,
           W1_self, b1_self, W1_in, W1_out, g1, be1,
           W2_self, b2_self, W2_in, W2_out, g2, be2,
           Wm1, bm1, Wm2, bm2, Wm3, bm3):
    f32 = jnp.float32
    src = edge_src.astype(jnp.int32)
    dst = edge_dst.astype(jnp.int32)
    row = lambda v: v.reshape(1, -1)

    bids_pad = jnp.pad(node_bank_ids.astype(jnp.int32), (0, NB_PAD - N))
    bank_pad = jnp.pad(bank_table, ((0, 0), (0, HALF - 16)))
    bepad = _bank_gather(bank_pad, bids_pad)

    ain2, aout2, abk2 = _segsum_l1(
        node_features.reshape(2 * N, HALF), bepad, src, dst)

    h = _combine1(node_features, bepad, ain2, abk2, aout2,
                  W1_self[:, :D].T, W1_self[:, D:].T,
                  W1_in[:, :D].T, W1_in[:, D:].T,
                  W1_out[:, :D].T, W1_out[:, D:].T,
                  row(b1_self), row(g1), row(be1))

    h2 = h.reshape(2 * N, HALF)
    a2in2, a2out2 = _segsum_pair(h2, src, dst)

    A = Wm1[:, 0:256]
    B = Wm1[:, 256:512]
    C = Wm1[:, 512:768]
    Dm = Wm1[:, 768:1024]
    Et = Wm1[:, 1024:1030].T
    FT = Wm1[:, 1030:1038].T
    GT = Wm1[:, 1038:1046].T
    HT = Wm1[:, 1046:1054].T
    hf, Woh = _combine2(h, a2in2, a2out2,
                        W2_self.T, W2_in.T, W2_out.T,
                        row(b2_self), row(g2), row(be2),
                        FT, GT, HT, currency_table, format_table)

    E2 = E // 2
    SHa, DHa = _edge_gather(hf, src[:E2], dst[:E2], E2)
    SHb, DHb = _edge_gather(hf, src[E2:], dst[E2:], E2)

    sm = jnp.concatenate([
        edge_numeric.astype(f32),
        jnp.ones((E, 1), f32),
        jnp.zeros((E, 1), f32),
        edge_sent_currency.astype(f32).reshape(E, 1),
        edge_recv_currency.astype(f32).reshape(E, 1),
        edge_payment_format.astype(f32).reshape(E, 1),
        jnp.zeros((E, 5), f32),
    ], axis=1)
    Wsm = jnp.concatenate([Et, bm1.reshape(1, D), jnp.zeros((9, D), f32)],
                          axis=0)
    Wcat5 = jnp.concatenate([A.T, B.T, C.T, Dm.T, Woh], axis=0)
    wm3 = Wm3.reshape(1, -1)
    args = (Wcat5, Wsm, Wm2.T, bm2.reshape(1, -1), wm3, bm3.reshape(1, 1))
    out_a = _edge_mlp(SHa, DHa, sm[:E2], *args, E2)
    out_b = _edge_mlp(SHb, DHb, sm[E2:], *args, E2)
    return jnp.concatenate([out_a[:, 0], out_b[:, 0]])
